# Initial kernel scaffold; baseline (speedup 1.0000x reference)
import jax
import jax.numpy as jnp
from jax.experimental import pallas as pl


def kernel(gt_points, gt_normals, gt_images, gt_depth, mask, pred_depth, reconst, pred_coord_0, pred_coord_1, pred_coord_2, pred_before_0, pred_before_1, pred_before_2, lap_idx_0, lap_idx_1, lap_idx_2, edges_0, edges_1, edges_2):
    raise NotImplementedError("write your pallas kernel here")


# speedup vs baseline: 5.7090x; 5.7090x over previous
import functools
import jax
import jax.numpy as jnp
from jax import lax
from jax.experimental import pallas as pl
from jax.experimental.pallas import tpu as pltpu
from jax.experimental.pallas import tpu_sc as plsc

_B = 4
_NG = 4096
_NS = (162, 642, 2562)
_ES = (480, 1920, 7680)
_NPS = (256, 768, 2688)
_STARTS = (0, 256, 1024)
_NPC = 3712
_VCS = (21, 81, 321)
_VCPS = (32, 96, 336)
_ECS = (60, 240, 960)
_ECPS = (64, 240, 960)
_G = 512
_NB = _NG // _G

_W_CHAMFER_OPP = 0.55
_W_LAPLACE = 0.5
_W_MOVE = 0.1
_W_EDGE = 0.1
_W_NORMAL = 0.00016
_W_RECONST = 0.1
_LAP_CONST = (0.2, 1.0, 1.0)


def _chamfer_body(gt_ref, pr_ref, sums_ref, d2_ref, i2_ref):
    nb = pl.program_id(1)
    first = nb == 0
    last = nb == _NB - 1
    gt = gt_ref[0]
    pr = pr_ref[0]
    gg = jnp.sum(gt * gt, axis=1, keepdims=True)
    pp = jnp.sum(pr * pr, axis=1)[None, :]
    m = lax.dot_general(gt, pr, (((1,), (1,)), ((), ())),
                        preferred_element_type=jnp.float32)
    d = gg + pp - 2.0 * m

    bmin = jnp.min(d, axis=0, keepdims=True)
    rows = lax.broadcasted_iota(jnp.int32, (_G, _NPC), 0) + nb * _G
    li = jnp.min(jnp.where(d == bmin, rows, jnp.int32(2 ** 30)),
                 axis=0, keepdims=True)
    prev_d = jnp.where(first, jnp.float32(3.0e38), d2_ref[0])
    prev_i = jnp.where(first, jnp.int32(0), i2_ref[0])
    upd = bmin < prev_d
    newd = jnp.where(upd, bmin, prev_d)
    newi = jnp.where(upd, li, prev_i)
    d2_ref[0] = newd
    i2_ref[0] = newi

    lane8 = lax.broadcasted_iota(jnp.int32, (1, 8), 1)
    io = lax.broadcasted_iota(jnp.int32, (1, _NPC), 1)
    sv = jnp.zeros((1, 8), jnp.float32)
    for l in range(3):
        s, np_, n = _STARTS[l], _NPS[l], _NS[l]
        s1 = jnp.sum(jnp.min(d[:, s:s + np_], axis=1, keepdims=True))
        sv = sv + jnp.where(lane8 == l, s1, 0.0)
        mk = (io >= s) & (io < s + n)
        s2 = jnp.sum(jnp.where(mk, newd, 0.0))
        sv = sv + jnp.where((lane8 == 3 + l) & last, s2, 0.0)
    prev_s = jnp.where(first, 0.0, sums_ref[0])
    sums_ref[0] = prev_s + sv


def _chamfer_call(gt_points, pred_cat):
    return pl.pallas_call(
        _chamfer_body,
        grid=(_B, _NB),
        in_specs=[
            pl.BlockSpec((1, _G, 3), lambda b, nb: (b, nb, 0)),
            pl.BlockSpec((1, _NPC, 3), lambda b, nb: (b, 0, 0)),
        ],
        out_specs=[
            pl.BlockSpec((1, 1, 8), lambda b, nb: (b, 0, 0)),
            pl.BlockSpec((1, 1, _NPC), lambda b, nb: (b, 0, 0)),
            pl.BlockSpec((1, 1, _NPC), lambda b, nb: (b, 0, 0)),
        ],
        out_shape=[
            jax.ShapeDtypeStruct((_B, 1, 8), jnp.float32),
            jax.ShapeDtypeStruct((_B, 1, _NPC), jnp.float32),
            jax.ShapeDtypeStruct((_B, 1, _NPC), jnp.int32),
        ],
    )(gt_points, pred_cat)


def _dense_body(gi_ref, rc_ref, gd_ref, pd_ref, mk_ref, out_ref):
    p = jnp.clip(rc_ref[...], 1e-7, 1.0 - 1e-7)
    gi = gi_ref[...]
    bce = jnp.sum(gi * jnp.log(p) + (1.0 - gi) * jnp.log(1.0 - p))
    m = (mk_ref[...] > 0.5).astype(jnp.float32)
    dlt = pd_ref[...] - gd_ref[...]
    ad = jnp.abs(dlt)
    sl1 = jnp.where(ad < 1.0, 0.5 * dlt * dlt, ad - 0.5)
    dnum = jnp.sum(sl1 * m)
    dden = jnp.sum(m)
    lane = lax.broadcasted_iota(jnp.int32, (1, 8), 1)
    out_ref[...] = (jnp.where(lane == 0, bce, 0.0)
                    + jnp.where(lane == 1, dnum, 0.0)
                    + jnp.where(lane == 2, dden, 0.0))


def _dense_call(gi, rc, gd, pd, mk):
    return pl.pallas_call(
        _dense_body,
        out_shape=jax.ShapeDtypeStruct((1, 8), jnp.float32),
    )(gi, rc, gd, pd, mk)


def _rsqrt16(x):
    i = plsc.bitcast(x, jnp.int32)
    i = jnp.int32(0x5F3759DF) - (i >> 1)
    y = plsc.bitcast(i, jnp.float32)
    for _ in range(3):
        y = y * (1.5 - 0.5 * x * y * y)
    return y


def _sc_body(px_h, py_h, pz_h, bx_h, by_h, bz_h, nx_h, ny_h, nz_h, i2_h,
             nbr0_h, nw0_h, sf0_h, wv0_h, rc0_h, a00_h, a10_h,
             nbr1_h, nw1_h, sf1_h, wv1_h, rc1_h, a01_h, a11_h,
             nbr2_h, nw2_h, sf2_h, wv2_h, rc2_h, a02_h, a12_h,
             out_h,
             px_t, py_t, pz_t, dx_t, dy_t, dz_t,
             nx_t, ny_t, nz_t, i2_t,
             nbr_t, nw_t, sf_t, wv_t, rc_t, a0_t, a1_t, out_t):
    wid = lax.axis_index("s") * 2 + lax.axis_index("c")
    b = wid // 8
    ch = wid % 8
    nbrs = (nbr0_h, nbr1_h, nbr2_h)
    nws = (nw0_h, nw1_h, nw2_h)
    sfs = (sf0_h, sf1_h, sf2_h)
    wvs = (wv0_h, wv1_h, wv2_h)
    rcs = (rc0_h, rc1_h, rc2_h)
    a0s = (a00_h, a01_h, a02_h)
    a1s = (a10_h, a11_h, a12_h)

    pltpu.sync_copy(nx_h.at[pl.ds(b * _NG, _NG)], nx_t)
    pltpu.sync_copy(ny_h.at[pl.ds(b * _NG, _NG)], ny_t)
    pltpu.sync_copy(nz_h.at[pl.ds(b * _NG, _NG)], nz_t)

    zero16 = jnp.zeros((16,), jnp.float32)
    for lvl in range(3):
        s, np_ = _STARTS[lvl], _NPS[lvl]
        vcp, ecp = _VCPS[lvl], _ECPS[lvl]
        po = b * _NPC + s
        pltpu.sync_copy(px_h.at[pl.ds(po, np_)], px_t.at[pl.ds(0, np_)])
        pltpu.sync_copy(py_h.at[pl.ds(po, np_)], py_t.at[pl.ds(0, np_)])
        pltpu.sync_copy(pz_h.at[pl.ds(po, np_)], pz_t.at[pl.ds(0, np_)])
        pltpu.sync_copy(bx_h.at[pl.ds(po, np_)], dx_t.at[pl.ds(0, np_)])
        pltpu.sync_copy(by_h.at[pl.ds(po, np_)], dy_t.at[pl.ds(0, np_)])
        pltpu.sync_copy(bz_h.at[pl.ds(po, np_)], dz_t.at[pl.ds(0, np_)])
        pltpu.sync_copy(i2_h.at[pl.ds(po, np_)], i2_t.at[pl.ds(0, np_)])
        pltpu.sync_copy(nbrs[lvl].at[pl.ds(ch * 8 * vcp, 8 * vcp)],
                        nbr_t.at[pl.ds(0, 8 * vcp)])
        pltpu.sync_copy(nws[lvl].at[pl.ds(ch * 8 * vcp, 8 * vcp)],
                        nw_t.at[pl.ds(0, 8 * vcp)])
        pltpu.sync_copy(sfs[lvl].at[pl.ds(ch * vcp, vcp)],
                        sf_t.at[pl.ds(0, vcp)])
        pltpu.sync_copy(wvs[lvl].at[pl.ds(ch * vcp, vcp)],
                        wv_t.at[pl.ds(0, vcp)])
        pltpu.sync_copy(rcs[lvl].at[pl.ds(ch * vcp, vcp)],
                        rc_t.at[pl.ds(0, vcp)])
        pltpu.sync_copy(a0s[lvl].at[pl.ds(ch * ecp, ecp)],
                        a0_t.at[pl.ds(0, ecp)])
        pltpu.sync_copy(a1s[lvl].at[pl.ds(ch * ecp, ecp)],
                        a1_t.at[pl.ds(0, ecp)])

        def dbody(g, c):
            off = g * 16
            dx_t[pl.ds(off, 16)] = dx_t[pl.ds(off, 16)] - px_t[pl.ds(off, 16)]
            dy_t[pl.ds(off, 16)] = dy_t[pl.ds(off, 16)] - py_t[pl.ds(off, 16)]
            dz_t[pl.ds(off, 16)] = dz_t[pl.ds(off, 16)] - pz_t[pl.ds(off, 16)]
            return c
        lax.fori_loop(0, np_ // 16, dbody, 0)

        def vbody(g, carry):
            lap_a, mv_a = carry
            off = g * 16
            sidx = sf_t[pl.ds(off, 16)]
            wv = wv_t[pl.ds(off, 16)]
            rc = rc_t[pl.ds(off, 16)]
            sx = plsc.load_gather(dx_t, [sidx])
            sy = plsc.load_gather(dy_t, [sidx])
            sz = plsc.load_gather(dz_t, [sidx])
            ax = zero16
            ay = zero16
            az = zero16
            for k in range(8):
                ko = k * vcp + off
                nk = nbr_t[pl.ds(ko, 16)]
                wk = nw_t[pl.ds(ko, 16)]
                ax = ax + wk * plsc.load_gather(dx_t, [nk])
                ay = ay + wk * plsc.load_gather(dy_t, [nk])
                az = az + wk * plsc.load_gather(dz_t, [nk])
            lx = (sx - ax * rc) * wv
            ly = (sy - ay * rc) * wv
            lz = (sz - az * rc) * wv
            lap_a = lap_a + lx * lx + ly * ly + lz * lz
            mv_a = mv_a + (sx * sx + sy * sy + sz * sz) * wv
            return (lap_a, mv_a)
        lap_v, mv_v = lax.fori_loop(0, vcp // 16, vbody, (zero16, zero16))

        def ebody(g, carry):
            eg_a, nr_a = carry
            off = g * 16
            a0v = a0_t[pl.ds(off, 16)]
            a1v = a1_t[pl.ds(off, 16)]
            dex = plsc.load_gather(px_t, [a0v]) - plsc.load_gather(px_t, [a1v])
            dey = plsc.load_gather(py_t, [a0v]) - plsc.load_gather(py_t, [a1v])
            dez = plsc.load_gather(pz_t, [a0v]) - plsc.load_gather(pz_t, [a1v])
            se = dex * dex + dey * dey + dez * dez
            i2v = plsc.load_gather(i2_t, [a0v])
            nxv = plsc.load_gather(nx_t, [i2v])
            nyv = plsc.load_gather(ny_t, [i2v])
            nzv = plsc.load_gather(nz_t, [i2v])
            dp = dex * nxv + dey * nyv + dez * nzv
            sn = nxv * nxv + nyv * nyv + nzv * nzv
            rse = _rsqrt16(jnp.maximum(se, 1e-24))
            rsn = _rsqrt16(jnp.maximum(sn, 1e-24))
            return (eg_a + se, nr_a + jnp.abs(dp) * rse * rsn)
        eg_v, nr_v = lax.fori_loop(0, ecp // 16, ebody, (zero16, zero16))

        out_t[pl.ds((4 * lvl + 0) * 16, 16)] = lap_v
        out_t[pl.ds((4 * lvl + 1) * 16, 16)] = mv_v
        out_t[pl.ds((4 * lvl + 2) * 16, 16)] = eg_v
        out_t[pl.ds((4 * lvl + 3) * 16, 16)] = nr_v
    for r in range(12, 16):
        out_t[pl.ds(r * 16, 16)] = zero16
    pltpu.sync_copy(out_t, out_h.at[pl.ds(wid * 256, 256)])


def _sc_call(args):
    mesh = plsc.VectorSubcoreMesh(core_axis_name="c", subcore_axis_name="s")
    f = pl.kernel(
        _sc_body,
        out_type=jax.ShapeDtypeStruct((8192,), jnp.float32),
        mesh=mesh,
        compiler_params=pltpu.CompilerParams(needs_layout_passes=False),
        scratch_types=[
            pltpu.VMEM((2688,), jnp.float32),
            pltpu.VMEM((2688,), jnp.float32),
            pltpu.VMEM((2688,), jnp.float32),
            pltpu.VMEM((2688,), jnp.float32),
            pltpu.VMEM((2688,), jnp.float32),
            pltpu.VMEM((2688,), jnp.float32),
            pltpu.VMEM((4096,), jnp.float32),
            pltpu.VMEM((4096,), jnp.float32),
            pltpu.VMEM((4096,), jnp.float32),
            pltpu.VMEM((2688,), jnp.int32),
            pltpu.VMEM((2688,), jnp.int32),
            pltpu.VMEM((2688,), jnp.float32),
            pltpu.VMEM((336,), jnp.int32),
            pltpu.VMEM((336,), jnp.float32),
            pltpu.VMEM((336,), jnp.float32),
            pltpu.VMEM((960,), jnp.int32),
            pltpu.VMEM((960,), jnp.int32),
            pltpu.VMEM((256,), jnp.float32),
        ],
    )
    return f(*args)


def _topo(lap_idx, edges, lvl):
    n, vc, vcp = _NS[lvl], _VCS[lvl], _VCPS[lvl]
    ec, ecp = _ECS[lvl], _ECPS[lvl]
    neigh = lap_idx[:, :8]
    cnt = lap_idx[:, 9]
    pad = 8 * vc - n
    neigh = jnp.pad(neigh, ((0, pad), (0, 0)), constant_values=-1)
    cntf = jnp.pad(cnt, (0, pad), constant_values=1).astype(jnp.float32)
    nb3 = neigh.reshape(8, vc, 8)
    nb3 = jnp.pad(nb3, ((0, 0), (0, vcp - vc), (0, 0)), constant_values=-1)
    nbr = jnp.transpose(nb3, (0, 2, 1))
    nw = (nbr >= 0).astype(jnp.float32)
    nbr = jnp.where(nbr < 0, 0, nbr)
    ids = jnp.arange(8 * vc, dtype=jnp.int32).reshape(8, vc)
    ids = jnp.pad(ids, ((0, 0), (0, vcp - vc)), constant_values=n)
    valid = ids < n
    sf = jnp.where(valid, ids, 0)
    wv = valid.astype(jnp.float32)
    rc = 1.0 / jnp.pad(cntf.reshape(8, vc), ((0, 0), (0, vcp - vc)),
                       constant_values=1.0)
    a0 = jnp.pad(edges[:, 0].reshape(8, ec), ((0, 0), (0, ecp - ec)))
    a1 = jnp.pad(edges[:, 1].reshape(8, ec), ((0, 0), (0, ecp - ec)))
    return (nbr.reshape(-1), nw.reshape(-1).astype(jnp.float32),
            sf.reshape(-1), wv.reshape(-1), rc.reshape(-1),
            a0.reshape(-1), a1.reshape(-1))


@jax.jit
def kernel(gt_points, gt_normals, gt_images, gt_depth, mask, pred_depth,
           reconst, pred_coord_0, pred_coord_1, pred_coord_2,
           pred_before_0, pred_before_1, pred_before_2,
           lap_idx_0, lap_idx_1, lap_idx_2, edges_0, edges_1, edges_2):
    pcs = (pred_coord_0, pred_coord_1, pred_coord_2)
    pbs = (pred_before_0, pred_before_1, pred_before_2)
    pc_pad = [jnp.pad(p, ((0, 0), (0, _NPS[i] - _NS[i]), (0, 0)),
                      constant_values=1e9) for i, p in enumerate(pcs)]
    pb_pad = [jnp.pad(p, ((0, 0), (0, _NPS[i] - _NS[i]), (0, 0)),
                      constant_values=1e9) for i, p in enumerate(pbs)]
    pred_cat = jnp.concatenate(pc_pad, axis=1)
    pb_cat = jnp.concatenate(pb_pad, axis=1)

    sums, _d2, idx2 = _chamfer_call(gt_points, pred_cat)
    sums = sums.reshape(_B, 8)
    idx2 = idx2.reshape(_B, _NPC)

    dense = _dense_call(
        gt_images.reshape(12, 50176), reconst.reshape(12, 50176),
        gt_depth.reshape(4, 50176), pred_depth.reshape(4, 50176),
        mask.reshape(4, 50176))

    sc_args = [pred_cat[:, :, 0].reshape(-1), pred_cat[:, :, 1].reshape(-1),
               pred_cat[:, :, 2].reshape(-1),
               pb_cat[:, :, 0].reshape(-1), pb_cat[:, :, 1].reshape(-1),
               pb_cat[:, :, 2].reshape(-1),
               gt_normals[:, :, 0].reshape(-1), gt_normals[:, :, 1].reshape(-1),
               gt_normals[:, :, 2].reshape(-1),
               idx2.reshape(-1)]
    laps = (lap_idx_0, lap_idx_1, lap_idx_2)
    edgs = (edges_0, edges_1, edges_2)
    for lvl in range(3):
        sc_args.extend(_topo(laps[lvl], edgs[lvl], lvl))
    sc_out = _sc_call(sc_args).reshape(32, 16, 16)
    q = jnp.sum(sc_out, axis=(0, 2))

    chamfer_loss = 0.0
    lap_loss = 0.0
    move_loss = 0.0
    edge_loss = 0.0
    normal_loss = 0.0
    for l in range(3):
        n = jnp.float32(_NS[l])
        e = jnp.float32(_ES[l])
        chamfer_loss = chamfer_loss + (jnp.sum(sums[:, l]) / _NG
                                       + _W_CHAMFER_OPP * jnp.sum(sums[:, 3 + l]) / n)
        lap_loss = lap_loss + _LAP_CONST[l] * q[4 * l + 0] / n
        if l > 0:
            move_loss = move_loss + _LAP_CONST[l] * q[4 * l + 1] / n
        edge_loss = edge_loss + q[4 * l + 2] / e
        normal_loss = normal_loss + q[4 * l + 3] / e
    image_loss = -dense[0, 0] / jnp.float32(12 * 50176)
    depth_loss = dense[0, 1] / jnp.maximum(dense[0, 2], 1.0)
    loss = (chamfer_loss + image_loss * _W_RECONST + _W_LAPLACE * lap_loss
            + _W_MOVE * move_loss + _W_EDGE * edge_loss
            + _W_NORMAL * normal_loss + depth_loss)
    return loss


# speedup vs baseline: 6.0601x; 1.0615x over previous
import functools
import jax
import jax.numpy as jnp
from jax import lax
from jax.experimental import pallas as pl
from jax.experimental.pallas import tpu as pltpu
from jax.experimental.pallas import tpu_sc as plsc

_B = 4
_NG = 4096
_NS = (162, 642, 2562)
_ES = (480, 1920, 7680)
_NPS = (256, 768, 2688)
_STARTS = (0, 256, 1024)
_NPC = 3712
_VCS = (21, 81, 321)
_VCPS = (32, 96, 336)
_ECS = (60, 240, 960)
_ECPS = (64, 240, 960)
_G = 512
_NB = _NG // _G

_W_CHAMFER_OPP = 0.55
_W_LAPLACE = 0.5
_W_MOVE = 0.1
_W_EDGE = 0.1
_W_NORMAL = 0.00016
_W_RECONST = 0.1
_LAP_CONST = (0.2, 1.0, 1.0)


def _chamfer_body(gt_ref, pr_ref, sums_ref, d2_ref, i2_ref):
    nb = pl.program_id(1)
    first = nb == 0
    last = nb == _NB - 1
    gt = gt_ref[0]
    pr = pr_ref[0]
    gg = jnp.sum(gt * gt, axis=1, keepdims=True)
    pp = jnp.sum(pr * pr, axis=1)[None, :]
    m2 = lax.dot_general(gt * -2.0, pr, (((1,), (1,)), ((), ())),
                         preferred_element_type=jnp.float32)
    d = (gg + pp) + m2

    bmin = jnp.min(d, axis=0, keepdims=True)
    rows = lax.broadcasted_iota(jnp.int32, (_G, _NPC), 0) + nb * _G
    li = jnp.min(jnp.where(d == bmin, rows, jnp.int32(2 ** 30)),
                 axis=0, keepdims=True)
    prev_d = jnp.where(first, jnp.float32(3.0e38), d2_ref[0])
    prev_i = jnp.where(first, jnp.int32(0), i2_ref[0])
    upd = bmin < prev_d
    newd = jnp.where(upd, bmin, prev_d)
    newi = jnp.where(upd, li, prev_i)
    d2_ref[0] = newd
    i2_ref[0] = newi

    lane8 = lax.broadcasted_iota(jnp.int32, (1, 8), 1)
    sv = jnp.zeros((1, 8), jnp.float32)
    for l in range(3):
        s, np_ = _STARTS[l], _NPS[l]
        s1 = jnp.sum(jnp.min(d[:, s:s + np_], axis=1, keepdims=True))
        sv = sv + jnp.where(lane8 == l, s1, 0.0)
    prev_s = jnp.where(first, 0.0, sums_ref[0])
    sums_ref[0] = prev_s + sv

    @pl.when(last)
    def _():
        io = lax.broadcasted_iota(jnp.int32, (1, _NPC), 1)
        sv2 = jnp.zeros((1, 8), jnp.float32)
        for l in range(3):
            s, n = _STARTS[l], _NS[l]
            mk = (io >= s) & (io < s + n)
            s2 = jnp.sum(jnp.where(mk, newd, 0.0))
            sv2 = sv2 + jnp.where(lane8 == 3 + l, s2, 0.0)
        sums_ref[0] = sums_ref[0] + sv2


def _chamfer_call(gt_points, pred_cat):
    return pl.pallas_call(
        _chamfer_body,
        grid=(_B, _NB),
        in_specs=[
            pl.BlockSpec((1, _G, 3), lambda b, nb: (b, nb, 0)),
            pl.BlockSpec((1, _NPC, 3), lambda b, nb: (b, 0, 0)),
        ],
        out_specs=[
            pl.BlockSpec((1, 1, 8), lambda b, nb: (b, 0, 0)),
            pl.BlockSpec((1, 1, _NPC), lambda b, nb: (b, 0, 0)),
            pl.BlockSpec((1, 1, _NPC), lambda b, nb: (b, 0, 0)),
        ],
        out_shape=[
            jax.ShapeDtypeStruct((_B, 1, 8), jnp.float32),
            jax.ShapeDtypeStruct((_B, 1, _NPC), jnp.float32),
            jax.ShapeDtypeStruct((_B, 1, _NPC), jnp.int32),
        ],
    )(gt_points, pred_cat)


def _dense_body(gi_ref, rc_ref, gd_ref, pd_ref, mk_ref, out_ref):
    p = jnp.clip(rc_ref[...], 1e-7, 1.0 - 1e-7)
    gi = gi_ref[...]
    bce = jnp.sum(gi * jnp.log(p) + (1.0 - gi) * jnp.log(1.0 - p))
    m = (mk_ref[...] > 0.5).astype(jnp.float32)
    dlt = pd_ref[...] - gd_ref[...]
    ad = jnp.abs(dlt)
    sl1 = jnp.where(ad < 1.0, 0.5 * dlt * dlt, ad - 0.5)
    dnum = jnp.sum(sl1 * m)
    dden = jnp.sum(m)
    lane = lax.broadcasted_iota(jnp.int32, (1, 8), 1)
    out_ref[...] = (jnp.where(lane == 0, bce, 0.0)
                    + jnp.where(lane == 1, dnum, 0.0)
                    + jnp.where(lane == 2, dden, 0.0))


def _dense_call(gi, rc, gd, pd, mk):
    return pl.pallas_call(
        _dense_body,
        out_shape=jax.ShapeDtypeStruct((1, 8), jnp.float32),
    )(gi, rc, gd, pd, mk)


def _rsqrt16(x):
    i = plsc.bitcast(x, jnp.int32)
    i = jnp.int32(0x5F3759DF) - (i >> 1)
    y = plsc.bitcast(i, jnp.float32)
    for _ in range(3):
        y = y * (1.5 - 0.5 * x * y * y)
    return y


def _sc_body(*refs):
    (px_h, py_h, pz_h, bx_h, by_h, bz_h, nx_h, ny_h, nz_h, i2_h) = refs[:10]
    out_h = refs[31]
    sc = refs[32:]
    sem = sc[-1]
    out_t = sc[-2]
    nrm_t = sc[21:24]
    wid = lax.axis_index("s") * 2 + lax.axis_index("c")
    b = wid // 8
    ch = wid % 8

    cps = []
    for i, h in enumerate((nx_h, ny_h, nz_h)):
        cps.append(pltpu.async_copy(h.at[pl.ds(b * _NG, _NG)], nrm_t[i], sem))
    for lvl in range(3):
        s, np_ = _STARTS[lvl], _NPS[lvl]
        vcp, ecp = _VCPS[lvl], _ECPS[lvl]
        pt = sc[7 * lvl:7 * lvl + 7]
        tp = sc[24 + 7 * lvl:31 + 7 * lvl]
        th = refs[10 + 7 * lvl:17 + 7 * lvl]
        po = b * _NPC + s
        for i, h in enumerate((px_h, py_h, pz_h, bx_h, by_h, bz_h, i2_h)):
            cps.append(pltpu.async_copy(h.at[pl.ds(po, np_)], pt[i], sem))
        offs = (ch * 8 * vcp, ch * 8 * vcp, ch * vcp, ch * vcp, ch * vcp,
                ch * ecp, ch * ecp)
        lens = (8 * vcp, 8 * vcp, vcp, vcp, vcp, ecp, ecp)
        for i in range(7):
            cps.append(pltpu.async_copy(th[i].at[pl.ds(offs[i], lens[i])],
                                        tp[i], sem))
    for c in cps:
        c.wait()

    zero16 = jnp.zeros((16,), jnp.float32)
    for lvl in range(3):
        s, np_ = _STARTS[lvl], _NPS[lvl]
        vcp, ecp = _VCPS[lvl], _ECPS[lvl]
        px_t, py_t, pz_t, dx_t, dy_t, dz_t, i2_t = sc[7 * lvl:7 * lvl + 7]
        nbr_t, nw_t, sf_t, wv_t, rc_t, a0_t, a1_t = sc[24 + 7 * lvl:31 + 7 * lvl]
        nx_t, ny_t, nz_t = nrm_t

        def dbody(g, c):
            off = g * 16
            dx_t[pl.ds(off, 16)] = dx_t[pl.ds(off, 16)] - px_t[pl.ds(off, 16)]
            dy_t[pl.ds(off, 16)] = dy_t[pl.ds(off, 16)] - py_t[pl.ds(off, 16)]
            dz_t[pl.ds(off, 16)] = dz_t[pl.ds(off, 16)] - pz_t[pl.ds(off, 16)]
            return c
        lax.fori_loop(0, np_ // 16, dbody, 0)

        def vbody(g, carry):
            lap_a, mv_a = carry
            off = g * 16
            sidx = sf_t[pl.ds(off, 16)]
            wv = wv_t[pl.ds(off, 16)]
            rc = rc_t[pl.ds(off, 16)]
            sx = plsc.load_gather(dx_t, [sidx])
            sy = plsc.load_gather(dy_t, [sidx])
            sz = plsc.load_gather(dz_t, [sidx])
            ax = zero16
            ay = zero16
            az = zero16
            for k in range(8):
                ko = k * vcp + off
                nk = nbr_t[pl.ds(ko, 16)]
                wk = nw_t[pl.ds(ko, 16)]
                ax = ax + wk * plsc.load_gather(dx_t, [nk])
                ay = ay + wk * plsc.load_gather(dy_t, [nk])
                az = az + wk * plsc.load_gather(dz_t, [nk])
            lx = (sx - ax * rc) * wv
            ly = (sy - ay * rc) * wv
            lz = (sz - az * rc) * wv
            lap_a = lap_a + lx * lx + ly * ly + lz * lz
            mv_a = mv_a + (sx * sx + sy * sy + sz * sz) * wv
            return (lap_a, mv_a)
        lap_v, mv_v = lax.fori_loop(0, vcp // 16, vbody, (zero16, zero16))

        def ebody(g, carry):
            eg_a, nr_a = carry
            off = g * 16
            a0v = a0_t[pl.ds(off, 16)]
            a1v = a1_t[pl.ds(off, 16)]
            dex = plsc.load_gather(px_t, [a0v]) - plsc.load_gather(px_t, [a1v])
            dey = plsc.load_gather(py_t, [a0v]) - plsc.load_gather(py_t, [a1v])
            dez = plsc.load_gather(pz_t, [a0v]) - plsc.load_gather(pz_t, [a1v])
            se = dex * dex + dey * dey + dez * dez
            i2v = plsc.load_gather(i2_t, [a0v])
            nxv = plsc.load_gather(nx_t, [i2v])
            nyv = plsc.load_gather(ny_t, [i2v])
            nzv = plsc.load_gather(nz_t, [i2v])
            dp = dex * nxv + dey * nyv + dez * nzv
            sn = nxv * nxv + nyv * nyv + nzv * nzv
            rse = _rsqrt16(jnp.maximum(se, 1e-24))
            rsn = _rsqrt16(jnp.maximum(sn, 1e-24))
            return (eg_a + se, nr_a + jnp.abs(dp) * rse * rsn)
        eg_v, nr_v = lax.fori_loop(0, ecp // 16, ebody, (zero16, zero16))

        out_t[pl.ds((4 * lvl + 0) * 16, 16)] = lap_v
        out_t[pl.ds((4 * lvl + 1) * 16, 16)] = mv_v
        out_t[pl.ds((4 * lvl + 2) * 16, 16)] = eg_v
        out_t[pl.ds((4 * lvl + 3) * 16, 16)] = nr_v
    for r in range(12, 16):
        out_t[pl.ds(r * 16, 16)] = zero16
    pltpu.sync_copy(out_t, out_h.at[pl.ds(wid * 256, 256)])


def _sc_call(args):
    mesh = plsc.VectorSubcoreMesh(core_axis_name="c", subcore_axis_name="s")
    f = pl.kernel(
        _sc_body,
        out_type=jax.ShapeDtypeStruct((8192,), jnp.float32),
        mesh=mesh,
        compiler_params=pltpu.CompilerParams(needs_layout_passes=False),
        scratch_types=(
            [t for lvl in range(3) for t in
             [pltpu.VMEM((_NPS[lvl],), jnp.float32)] * 6
             + [pltpu.VMEM((_NPS[lvl],), jnp.int32)]]
            + [pltpu.VMEM((_NG,), jnp.float32)] * 3
            + [t for lvl in range(3) for t in
               [pltpu.VMEM((8 * _VCPS[lvl],), jnp.int32),
                pltpu.VMEM((8 * _VCPS[lvl],), jnp.float32),
                pltpu.VMEM((_VCPS[lvl],), jnp.int32),
                pltpu.VMEM((_VCPS[lvl],), jnp.float32),
                pltpu.VMEM((_VCPS[lvl],), jnp.float32),
                pltpu.VMEM((_ECPS[lvl],), jnp.int32),
                pltpu.VMEM((_ECPS[lvl],), jnp.int32)]]
            + [pltpu.VMEM((256,), jnp.float32),
               pltpu.SemaphoreType.DMA]
        ),
    )
    return f(*args)


def _topo(lap_idx, edges, lvl):
    n, vc, vcp = _NS[lvl], _VCS[lvl], _VCPS[lvl]
    ec, ecp = _ECS[lvl], _ECPS[lvl]
    neigh = lap_idx[:, :8]
    cnt = lap_idx[:, 9]
    pad = 8 * vc - n
    neigh = jnp.pad(neigh, ((0, pad), (0, 0)), constant_values=-1)
    cntf = jnp.pad(cnt, (0, pad), constant_values=1).astype(jnp.float32)
    nb3 = neigh.reshape(8, vc, 8)
    nb3 = jnp.pad(nb3, ((0, 0), (0, vcp - vc), (0, 0)), constant_values=-1)
    nbr = jnp.transpose(nb3, (0, 2, 1))
    nw = (nbr >= 0).astype(jnp.float32)
    nbr = jnp.where(nbr < 0, 0, nbr)
    ids = jnp.arange(8 * vc, dtype=jnp.int32).reshape(8, vc)
    ids = jnp.pad(ids, ((0, 0), (0, vcp - vc)), constant_values=n)
    valid = ids < n
    sf = jnp.where(valid, ids, 0)
    wv = valid.astype(jnp.float32)
    rc = 1.0 / jnp.pad(cntf.reshape(8, vc), ((0, 0), (0, vcp - vc)),
                       constant_values=1.0)
    a0 = jnp.pad(edges[:, 0].reshape(8, ec), ((0, 0), (0, ecp - ec)))
    a1 = jnp.pad(edges[:, 1].reshape(8, ec), ((0, 0), (0, ecp - ec)))
    return (nbr.reshape(-1), nw.reshape(-1).astype(jnp.float32),
            sf.reshape(-1), wv.reshape(-1), rc.reshape(-1),
            a0.reshape(-1), a1.reshape(-1))


@jax.jit
def kernel(gt_points, gt_normals, gt_images, gt_depth, mask, pred_depth,
           reconst, pred_coord_0, pred_coord_1, pred_coord_2,
           pred_before_0, pred_before_1, pred_before_2,
           lap_idx_0, lap_idx_1, lap_idx_2, edges_0, edges_1, edges_2):
    pcs = (pred_coord_0, pred_coord_1, pred_coord_2)
    pbs = (pred_before_0, pred_before_1, pred_before_2)
    pc_pad = [jnp.pad(p, ((0, 0), (0, _NPS[i] - _NS[i]), (0, 0)),
                      constant_values=1e9) for i, p in enumerate(pcs)]
    pb_pad = [jnp.pad(p, ((0, 0), (0, _NPS[i] - _NS[i]), (0, 0)),
                      constant_values=1e9) for i, p in enumerate(pbs)]
    pred_cat = jnp.concatenate(pc_pad, axis=1)
    pb_cat = jnp.concatenate(pb_pad, axis=1)

    sums, _d2, idx2 = _chamfer_call(gt_points, pred_cat)
    sums = sums.reshape(_B, 8)
    idx2 = idx2.reshape(_B, _NPC)

    dense = _dense_call(
        gt_images.reshape(12, 50176), reconst.reshape(12, 50176),
        gt_depth.reshape(4, 50176), pred_depth.reshape(4, 50176),
        mask.reshape(4, 50176))

    sc_args = [pred_cat[:, :, 0].reshape(-1), pred_cat[:, :, 1].reshape(-1),
               pred_cat[:, :, 2].reshape(-1),
               pb_cat[:, :, 0].reshape(-1), pb_cat[:, :, 1].reshape(-1),
               pb_cat[:, :, 2].reshape(-1),
               gt_normals[:, :, 0].reshape(-1), gt_normals[:, :, 1].reshape(-1),
               gt_normals[:, :, 2].reshape(-1),
               idx2.reshape(-1)]
    laps = (lap_idx_0, lap_idx_1, lap_idx_2)
    edgs = (edges_0, edges_1, edges_2)
    for lvl in range(3):
        sc_args.extend(_topo(laps[lvl], edgs[lvl], lvl))
    sc_out = _sc_call(sc_args).reshape(32, 16, 16)
    q = jnp.sum(sc_out, axis=(0, 2))

    chamfer_loss = 0.0
    lap_loss = 0.0
    move_loss = 0.0
    edge_loss = 0.0
    normal_loss = 0.0
    for l in range(3):
        n = jnp.float32(_NS[l])
        e = jnp.float32(_ES[l])
        chamfer_loss = chamfer_loss + (jnp.sum(sums[:, l]) / _NG
                                       + _W_CHAMFER_OPP * jnp.sum(sums[:, 3 + l]) / n)
        lap_loss = lap_loss + _LAP_CONST[l] * q[4 * l + 0] / n
        if l > 0:
            move_loss = move_loss + _LAP_CONST[l] * q[4 * l + 1] / n
        edge_loss = edge_loss + q[4 * l + 2] / e
        normal_loss = normal_loss + q[4 * l + 3] / e
    image_loss = -dense[0, 0] / jnp.float32(12 * 50176)
    depth_loss = dense[0, 1] / jnp.maximum(dense[0, 2], 1.0)
    loss = (chamfer_loss + image_loss * _W_RECONST + _W_LAPLACE * lap_loss
            + _W_MOVE * move_loss + _W_EDGE * edge_loss
            + _W_NORMAL * normal_loss + depth_loss)
    return loss
